# baseline (speedup 1.0000x reference)
import numpy as np
import jax
import jax.numpy as jnp
from jax.experimental import pallas as pl

B = 8
DIM = 32
H = 32
W = 32
NE = 8192
NPIX = H * W
NCOL = B * NPIX
SCALE_LIST = (1, 2, 4, 8, 16, 32)
PHI_IDX = (0, 0, 1, 2, 3, 3)
SMALL = (1, 2)
CHUNK = 128
NCHUNK = NE // CHUNK
OFFS = tuple((dy, dx) for dy in (-1, 0, 1) for dx in (-1, 0, 1))


def _pool_mat(p):
    f = H // p
    m = np.zeros((NPIX, p * p), np.float32)
    for cy in range(p):
        for cx in range(p):
            for yy in range(f):
                for xx in range(f):
                    m[(cy * f + yy) * W + cx * f + xx, cy * p + cx] = 1.0
    return m


_POOLS = {}
for _p in SCALE_LIST[:-1]:
    _m = _pool_mat(_p)
    if _p in SMALL:
        _m = np.kron(np.eye(B, dtype=np.float32), _m)
    _POOLS[_p] = _m


def _resize_mats():
    m1t, m2t = {}, {}
    for p in SCALE_LIST[1:-1]:
        a = jax.image.resize(jnp.eye(p, dtype=jnp.float32), (H, p),
                             method='bicubic')
        ip = jnp.eye(p, dtype=jnp.float32)
        i32 = jnp.eye(H, dtype=jnp.float32)
        m1t[p] = jnp.kron(a, ip).T
        m2t[p] = jnp.kron(i32, a).T
    a1 = jax.image.resize(jnp.eye(1, dtype=jnp.float32), (H, 1),
                          method='bicubic')
    u1 = jnp.kron(a1, a1).T
    u1 = jnp.kron(jnp.eye(B, dtype=jnp.float32), u1)
    return m1t, m2t, u1


def _mm(a, b):
    return jnp.dot(a, b, preferred_element_type=jnp.float32,
                   precision=jax.lax.Precision.HIGHEST)


def _mm_bf(a, b):
    return jnp.dot(a.astype(jnp.bfloat16), b.astype(jnp.bfloat16),
                   preferred_element_type=jnp.float32)


def _body(z_ref, emb_ref, embT_ref,
          w9_0, w9_1, w9_2, w9_3, bi_0, bi_1, bi_2, bi_3,
          p1, p2, p4, p8, p16,
          m1_2, m1_4, m1_8, m1_16, m2_2, m2_4, m2_8, m2_16, u1,
          zhat_ref, loss_ref, i0, i1, i2, i3, i4, i5):
    w9 = (w9_0, w9_1, w9_2, w9_3)
    bias = (bi_0, bi_1, bi_2, bi_3)
    pools = {1: p1, 2: p2, 4: p4, 8: p8, 16: p16}
    m1s = {2: m1_2, 4: m1_4, 8: m1_8, 16: m1_16}
    m2s = {2: m2_2, 4: m2_4, 8: m2_8, 16: m2_16}
    idx_refs = (i0, i1, i2, i3, i4, i5)

    z = z_ref[...]
    col = jax.lax.broadcasted_iota(jnp.int32, (1, NCOL), 1)
    ycoord = (col // W) % H
    xcoord = col % W
    masks = []
    for dy, dx in OFFS:
        ok = jnp.ones((1, NCOL), jnp.bool_)
        if dy == -1:
            ok = ycoord >= 1
        elif dy == 1:
            ok = ycoord <= H - 2
        if dx == -1:
            ok = jnp.logical_and(ok, xcoord >= 1)
        elif dx == 1:
            ok = jnp.logical_and(ok, xcoord <= W - 2)
        masks.append(ok.astype(jnp.float32))

    def _shift(hh, s):
        if s == 0:
            return hh
        zpad = jnp.zeros((DIM, abs(s)), jnp.float32)
        if s > 0:
            return jnp.concatenate([hh[:, s:], zpad], axis=1)
        return jnp.concatenate([zpad, hh[:, :s]], axis=1)

    def _phi(hh, pi):
        rows = []
        for k, (dy, dx) in enumerate(OFFS):
            sh = _shift(hh, dy * W + dx)
            if not (dy == 0 and dx == 0):
                sh = sh * masks[k]
            rows.append(sh.astype(jnp.bfloat16))
        x9 = jnp.concatenate(rows, axis=0)
        conv = jnp.dot(w9[pi][...], x9,
                       preferred_element_type=jnp.float32) + bias[pi][...]
        return hh * 0.5 + conv * 0.5

    def _vq(r, n):
        rt = jnp.transpose(r)
        rr = jnp.sum(rt * rt, axis=1, keepdims=True)

        def body(c, carry):
            bd, bix, bv = carry
            ec = emb_ref[pl.ds(c * CHUNK, CHUNK), :]
            et = embT_ref[:, pl.ds(c * CHUNK, CHUNK)]
            ee = jnp.transpose(
                jnp.sum(ec * ec, axis=1, keepdims=True))
            s = _mm_bf(rt, et)
            dist = jnp.sqrt(jnp.maximum(rr + ee - 2.0 * s, 0.0))
            cmin = jnp.min(dist, axis=1, keepdims=True)
            io = jax.lax.broadcasted_iota(jnp.int32, (n, CHUNK), 1)
            carg = jnp.min(jnp.where(dist == cmin, io, CHUNK),
                           axis=1, keepdims=True)
            onehot = (io == carg).astype(jnp.float32)
            vec = _mm(onehot, ec)
            upd = cmin < bd
            bd = jnp.where(upd, cmin, bd)
            bix = jnp.where(upd, carg + c * CHUNK, bix)
            bv = jnp.where(jnp.broadcast_to(upd, (n, DIM)), vec, bv)
            return bd, bix, bv

        init = (jnp.full((n, 1), 1e30, jnp.float32),
                jnp.zeros((n, 1), jnp.int32),
                jnp.zeros((n, DIM), jnp.float32))
        _, bix, bv = jax.lax.fori_loop(0, NCHUNK, body, init)
        return bix, jnp.transpose(bv)

    def _vq_tiled(r, n):
        tile = 4096
        if n <= tile:
            return _vq(r, n)
        parts = [_vq(r[:, i * tile:(i + 1) * tile], tile)
                 for i in range(n // tile)]
        return (jnp.concatenate([pr[0] for pr in parts], axis=0),
                jnp.concatenate([pr[1] for pr in parts], axis=1))

    zres = z
    zhat = jnp.zeros_like(z)
    loss = jnp.zeros((1, 1), jnp.float32)
    for si, p in enumerate(SCALE_LIST):
        n = B * p * p
        pp = p * p
        f2inv = 1.0 / float((H // p) * (H // p)) if p != H else 1.0
        if p == H:
            r = zres
        elif p in SMALL:
            r = _mm(zres, pools[p][...]) * f2inv
        else:
            pm = pools[p][...]
            r = jnp.concatenate(
                [_mm(zres[:, b * NPIX:(b + 1) * NPIX], pm) for b in range(B)],
                axis=1) * f2inv
        bix, q = _vq_tiled(r, n)
        idx_refs[si][...] = bix
        if p == H:
            h = q
        elif p == 1:
            h = _mm(q, u1[...])
        else:
            m1 = m1s[p][...]
            m2 = m2s[p][...]
            h = jnp.concatenate(
                [_mm(_mm(q[:, b * pp:(b + 1) * pp], m1), m2)
                 for b in range(B)], axis=1)
        qp = _phi(h, PHI_IDX[si])
        zhat = zhat + qp
        zres = zres - qp
        loss = loss + jnp.sum((zhat - z) ** 2, keepdims=True).reshape(1, 1) * (
            1.25 / (DIM * NCOL))
    zhat_ref[...] = zhat
    loss_ref[...] = loss / 6.0


def kernel(z, emb, w0, w1, w2, w3, b0, b1, b2, b3):
    z_cm = jnp.transpose(z, (1, 0, 2, 3)).reshape(DIM, NCOL)
    embT = jnp.transpose(emb)
    w9 = [jnp.transpose(w, (2, 3, 1, 0)).reshape(9 * DIM, DIM).T.astype(
        jnp.bfloat16) for w in (w0, w1, w2, w3)]
    bias = [b.reshape(DIM, 1) for b in (b0, b1, b2, b3)]
    m1t, m2t, u1 = _resize_mats()
    out_shape = (
        [jax.ShapeDtypeStruct((DIM, NCOL), jnp.float32),
         jax.ShapeDtypeStruct((1, 1), jnp.float32)]
        + [jax.ShapeDtypeStruct((B * p * p, 1), jnp.int32) for p in SCALE_LIST])
    outs = pl.pallas_call(_body, out_shape=out_shape)(
        z_cm, emb, embT, *w9, *bias,
        *[_POOLS[p] for p in SCALE_LIST[:-1]],
        *[m1t[p] for p in (2, 4, 8, 16)],
        *[m2t[p] for p in (2, 4, 8, 16)], u1)
    zhat_cm, loss = outs[0], outs[1]
    idxs = outs[2:]
    z_hat = jnp.transpose(zhat_cm.reshape(DIM, B, NPIX), (1, 0, 2)).reshape(
        B, DIM, H, W)
    total_idx = jnp.concatenate(
        [ix.reshape(B, p * p) for ix, p in zip(idxs, SCALE_LIST)], axis=1)
    return z_hat, loss[0, 0], total_idx
